# baseline (speedup 1.0000x reference)
import functools

import jax
import jax.numpy as jnp
from jax import lax
from jax.experimental import pallas as pl
from jax.experimental.pallas import tpu as pltpu
from jax.experimental.pallas import tpu_sc as plsc

B = 4096
L = 200
D = 300
DA = 256
DB = 48
LANES = 16
NC = 2
NS = 16
NW = NC * NS
PPW = B // NW
OBLK = 16

PAD_BLK = 8000


def _tail_body(t_ref, o_ref):
    o_ref[...] = t_ref[...]


def _tail_table(table):
    v = table.shape[0]
    return pl.pallas_call(
        _tail_body,
        grid=(v // PAD_BLK,),
        in_specs=[pl.BlockSpec((PAD_BLK, 128), lambda i: (i, 2))],
        out_specs=pl.BlockSpec((PAD_BLK, 128), lambda i: (i, 0)),
        out_shape=jax.ShapeDtypeStruct((v, 128), jnp.float32),
    )(table)


def _make_body(nch, two_slices):
    def body(idx_hbm, tab_hbm, out_hbm, idx_v, rows_v, out_v, sem):
        wid = lax.axis_index("s") * NC + lax.axis_index("c")
        base = wid * PPW
        scale = jnp.float32(1.0 / L)
        zero = jnp.zeros((LANES,), jnp.float32)

        def per_post(p, carry):
            pltpu.sync_copy(idx_hbm.at[base + p], idx_v)
            if two_slices:
                c0 = pltpu.async_copy(tab_hbm.at[idx_v, pl.ds(0, 128)],
                                      rows_v.at[:, pl.ds(0, 128)], sem)
                c1 = pltpu.async_copy(tab_hbm.at[idx_v, pl.ds(128, 128)],
                                      rows_v.at[:, pl.ds(128, 128)], sem)
                c0.wait()
                c1.wait()
            else:
                pltpu.async_copy(tab_hbm.at[idx_v], rows_v, sem).wait()

            def per_row(t, accs):
                return tuple(accs[c] + rows_v[t, pl.ds(c * LANES, LANES)]
                             for c in range(nch))

            accs = lax.fori_loop(0, L, per_row, (zero,) * nch)
            pm = lax.rem(p, jnp.int32(OBLK))
            for c in range(nch):
                out_v[pm, pl.ds(c * LANES, LANES)] = accs[c] * scale

            @pl.when(pm == OBLK - 1)
            def _():
                start = pl.multiple_of(base + p - (OBLK - 1), OBLK)
                pltpu.sync_copy(out_v,
                                out_hbm.at[pl.ds(start, OBLK), :])

            return carry

        lax.fori_loop(0, PPW, per_post, 0)

    return body


def _make_kernel(dout, nch, rows_w, two_slices):
    mesh = plsc.VectorSubcoreMesh(core_axis_name="c", subcore_axis_name="s")
    return functools.partial(
        pl.kernel,
        mesh=mesh,
        compiler_params=pltpu.CompilerParams(use_tc_tiling_on_sc=True,
                                             needs_layout_passes=False),
        out_type=jax.ShapeDtypeStruct((B, dout), jnp.float32),
        scratch_types=[
            pltpu.VMEM((L,), jnp.int32),
            pltpu.VMEM((L, rows_w), jnp.float32),
            pltpu.VMEM((OBLK, dout), jnp.float32),
            pltpu.SemaphoreType.DMA,
        ],
    )(_make_body(nch, two_slices))


@jax.jit
def _run(indices, table):
    tail = _tail_table(table)
    out_a = _make_kernel(DA, DA // LANES, DA, True)(indices, table)
    out_b = _make_kernel(DB, DB // LANES, 128, False)(indices, tail)
    return jnp.concatenate([out_a, out_b[:, :D - DA]], axis=1)


def kernel(indices, table):
    return _run(indices, table)


# speedup vs baseline: 1.0884x; 1.0884x over previous
import functools

import jax
import jax.numpy as jnp
from jax import lax
from jax.experimental import pallas as pl
from jax.experimental.pallas import tpu as pltpu
from jax.experimental.pallas import tpu_sc as plsc

B = 4096
L = 200
D = 300
DP = 384
DO = 304
LANES = 16
NCH = DO // LANES
NC = 2
NS = 16
NW = NC * NS
PPW = B // NW
OBLK = 16


def _body(idx_hbm, tab_hbm, tail_hbm, out_hbm, idx_v, rows_v, out_v, sem):
    wid = lax.axis_index("s") * NC + lax.axis_index("c")
    base = wid * PPW
    scale = jnp.float32(1.0 / L)
    zero = jnp.zeros((LANES,), jnp.float32)

    def per_post(p, carry):
        pltpu.sync_copy(idx_hbm.at[base + p], idx_v)
        c0 = pltpu.async_copy(tab_hbm.at[idx_v, pl.ds(0, 128)],
                              rows_v.at[:, pl.ds(0, 128)], sem)
        c1 = pltpu.async_copy(tab_hbm.at[idx_v, pl.ds(128, 128)],
                              rows_v.at[:, pl.ds(128, 128)], sem)
        c2 = pltpu.async_copy(tail_hbm.at[idx_v],
                              rows_v.at[:, pl.ds(256, 128)], sem)
        c0.wait()
        c1.wait()
        c2.wait()

        def per_row(t, accs):
            return tuple(accs[c] + rows_v[t, pl.ds(c * LANES, LANES)]
                         for c in range(NCH))

        accs = lax.fori_loop(0, L, per_row, (zero,) * NCH)
        pm = lax.rem(p, jnp.int32(OBLK))
        for c in range(NCH):
            out_v[pm, pl.ds(c * LANES, LANES)] = accs[c] * scale

        @pl.when(pm == OBLK - 1)
        def _():
            start = pl.multiple_of(base + p - (OBLK - 1), OBLK)
            pltpu.sync_copy(out_v, out_hbm.at[pl.ds(start, OBLK), :])

        return carry

    lax.fori_loop(0, PPW, per_post, 0)


PAD_BLK = 8000


def _tail_body(t_ref, o_ref):
    o_ref[...] = t_ref[...]


def _tail_table(table):
    v = table.shape[0]
    return pl.pallas_call(
        _tail_body,
        grid=(v // PAD_BLK,),
        in_specs=[pl.BlockSpec((PAD_BLK, 128), lambda i: (i, 2))],
        out_specs=pl.BlockSpec((PAD_BLK, 128), lambda i: (i, 0)),
        out_shape=jax.ShapeDtypeStruct((v, 128), jnp.float32),
    )(table)


@jax.jit
def _run(indices, table):
    tail = _tail_table(table)
    mesh = plsc.VectorSubcoreMesh(core_axis_name="c", subcore_axis_name="s")
    kern = functools.partial(
        pl.kernel,
        mesh=mesh,
        compiler_params=pltpu.CompilerParams(use_tc_tiling_on_sc=True,
                                             needs_layout_passes=False),
        out_type=jax.ShapeDtypeStruct((B, DO), jnp.float32),
        scratch_types=[
            pltpu.VMEM((L,), jnp.int32),
            pltpu.VMEM((L, DP), jnp.float32),
            pltpu.VMEM((OBLK, DO), jnp.float32),
            pltpu.SemaphoreType.DMA,
        ],
    )(_body)
    return kern(indices, table, tail)[:, :D]


def kernel(indices, table):
    return _run(indices, table)
